# baseline (speedup 1.0000x reference)
import jax
import jax.numpy as jnp
from jax import lax
from jax.experimental import pallas as pl
from jax.experimental.pallas import tpu as pltpu
from jax.experimental.pallas import tpu_sc as plsc
from jax._src.pallas import mpmd

MEM = 10000
D = 3072
B = 1024
R = 512
NC = 2
NS = 16
NW = NC * NS
L = 16

_MESH = plsc.VectorSubcoreMesh(core_axis_name="c", subcore_axis_name="s")
_SC_PARAMS = pltpu.CompilerParams(needs_layout_passes=False)


_NCHAIN = 5
_CSTEPS = 10
_CBLK = MEM // (_NCHAIN * _CSTEPS)


def _copy_body(src_ref, dst_ref, vb, rsem, wsem):
    def _rd(q, step, p):
        blk = q * _CSTEPS + step
        return pltpu.make_async_copy(
            src_ref.at[pl.ds(blk * _CBLK, _CBLK)], vb.at[2 * q + p], rsem)

    def _wr(q, step, p):
        blk = q * _CSTEPS + step
        return pltpu.make_async_copy(
            vb.at[2 * q + p], dst_ref.at[pl.ds(blk * _CBLK, _CBLK)], wsem)

    for q in range(_NCHAIN):
        _rd(q, 0, 0).start()
    for step in range(_CSTEPS):
        p = step % 2
        for q in range(_NCHAIN):
            _rd(q, step, p).wait()
            if step >= 1:
                _wr(q, step - 1, 1 - p).wait()
            if step + 1 < _CSTEPS:
                _rd(q, step + 1, 1 - p).start()
            _wr(q, step, p).start()
    for q in range(_NCHAIN):
        _wr(q, _CSTEPS - 1, (_CSTEPS - 1) % 2).wait()


@jax.jit
def _tc_copy(buf):
    return pl.pallas_call(
        _copy_body,
        out_shape=jax.ShapeDtypeStruct((MEM, D), jnp.float32),
        in_specs=[pl.BlockSpec(memory_space=pl.ANY)],
        out_specs=pl.BlockSpec(memory_space=pl.ANY),
        scratch_shapes=[
            pltpu.VMEM((2 * _NCHAIN, _CBLK, D), jnp.float32),
            pltpu.SemaphoreType.DMA,
            pltpu.SemaphoreType.DMA,
        ],
    )(buf)


def _update_body(img_in, x, y, idx, blab, ridx,
                 img_out, nlab, rimg, rlab,
                 idx_v, w_v, stage, ridx_v, lab_v, y_v, ridx_all, rlab_v,
                 w_sh,
                 sem, ssem):
    c = lax.axis_index("c")
    s = lax.axis_index("s")
    wid = s * NC + c
    lanes = lax.iota(jnp.int32, L)

    pltpu.sync_copy(idx, idx_v)

    @pl.when(s == 0)
    def _build_map():
        def mset(i, carry):
            w_v[pl.ds(i * L, L)] = jnp.full((L,), -1, jnp.int32)
            return carry

        lax.fori_loop(0, MEM // L, mset, 0)

        def setw(ci, carry):
            iv = idx_v[pl.ds(ci * L, L)]
            bids = ci * L + lanes
            for l in range(L):
                plsc.store_scatter(w_v, [iv], bids, mask=lanes == l)
            return carry

        lax.fori_loop(0, B // L, setw, 0)
        pltpu.sync_copy(w_v, w_sh)

    plsc.subcore_barrier()
    pltpu.sync_copy(w_sh, w_v)

    per = B // NW
    base = wid * per
    for h in range(per // L):
        dv = idx_v[pl.ds(base + h * L, L)]
        srcs = plsc.load_gather(w_v, [dv])
        pltpu.async_copy(x.at[srcs], stage.at[h], sem).wait()
        pltpu.async_copy(stage.at[h], img_out.at[dv], sem).wait()

    rbase = wid * (R // NW)
    pltpu.sync_copy(ridx.at[pl.ds(rbase, R // NW)], ridx_v)
    rv = ridx_v[...]
    pltpu.async_copy(img_in.at[rv], stage.at[0], ssem).wait()
    pltpu.async_copy(stage.at[0], rimg.at[pl.ds(rbase, R // NW)], ssem).wait()
    wr = plsc.load_gather(w_v, [rv])
    updr = wr >= 0

    @pl.when(jnp.any(updr))
    def _ret_patch():
        f = plsc.all_reduce_ffs(updr)
        rv_f = plsc.load_gather(ridx_v, [f])
        wr_f = plsc.load_gather(w_v, [rv_f])
        srcs = jnp.where(updr, wr, wr_f)
        dst = jnp.where(updr, rbase + lanes, rbase + f)
        pltpu.async_copy(x.at[srcs], stage.at[1], ssem).wait()
        pltpu.async_copy(stage.at[1], rimg.at[dst], ssem).wait()

    @pl.when(jnp.logical_and(s == 0, c == 0))
    def _labels():
        pltpu.sync_copy(blab, lab_v)
        pltpu.sync_copy(y, y_v)

        def setl(ci, carry):
            iv = idx_v[pl.ds(ci * L, L)]
            yv = y_v[pl.ds(ci * L, L)]
            for l in range(L):
                plsc.store_scatter(lab_v, [iv], yv, mask=lanes == l)
            return carry

        lax.fori_loop(0, B // L, setl, 0)
        pltpu.sync_copy(lab_v, nlab)

        pltpu.sync_copy(ridx, ridx_all)

        def rl(ci, carry):
            rr = ridx_all[pl.ds(ci * L, L)]
            rlab_v[pl.ds(ci * L, L)] = plsc.load_gather(lab_v, [rr])
            return carry

        lax.fori_loop(0, R // L, rl, 0)
        pltpu.sync_copy(rlab_v, rlab)


_sc_update = mpmd._mpmd_map(
    [(_MESH, _update_body)],
    out_types=[
        jax.ShapeDtypeStruct((MEM, D), jnp.float32),
        jax.ShapeDtypeStruct((MEM,), jnp.int32),
        jax.ShapeDtypeStruct((R, D), jnp.float32),
        jax.ShapeDtypeStruct((R,), jnp.int32),
    ],
    input_output_aliases={0: 0},
    compiler_params=_SC_PARAMS,
    scratch_types=[
        pltpu.VMEM((B,), jnp.int32),
        pltpu.VMEM((MEM,), jnp.int32),
        pltpu.VMEM((2, L, D), jnp.float32),
        pltpu.VMEM((R // NW,), jnp.int32),
        pltpu.VMEM((MEM,), jnp.int32),
        pltpu.VMEM((B,), jnp.int32),
        pltpu.VMEM((R,), jnp.int32),
        pltpu.VMEM((R,), jnp.int32),
        pltpu.VMEM_SHARED((MEM,), jnp.int32),
        pltpu.SemaphoreType.DMA,
        pltpu.SemaphoreType.DMA,
    ],
)


def kernel(buffer_img, buffer_label, x, y, idx, retrieve_idx):
    img2 = buffer_img.reshape(MEM, D)
    x2 = x.reshape(B, D)
    y32 = y.astype(jnp.int32)
    idx32 = idx.astype(jnp.int32)
    ridx32 = retrieve_idx.astype(jnp.int32)
    blab32 = buffer_label.astype(jnp.int32)

    img0 = _tc_copy(img2)
    new_img2, new_label, ret_img2, ret_label = _sc_update(
        img0, x2, y32, idx32, blab32, ridx32)

    new_img = new_img2.reshape(MEM, 3, 32, 32)
    ret_img = ret_img2.reshape(R, 3, 32, 32)
    return (new_img,
            new_label.astype(buffer_label.dtype),
            ret_img,
            ret_label.astype(buffer_label.dtype))


# speedup vs baseline: 1.0234x; 1.0234x over previous
import jax
import jax.numpy as jnp
from jax import lax
from jax.experimental import pallas as pl
from jax.experimental.pallas import tpu as pltpu
from jax.experimental.pallas import tpu_sc as plsc
from jax._src.pallas import mpmd

MEM = 10000
D = 3072
B = 1024
R = 512
NC = 2
NS = 16
NW = NC * NS
L = 16

_MESH = plsc.VectorSubcoreMesh(core_axis_name="c", subcore_axis_name="s")
_SC_PARAMS = pltpu.CompilerParams(needs_layout_passes=False)


_NCHAIN = 5
_CSTEPS = 10
_CBLK = MEM // (_NCHAIN * _CSTEPS)


def _copy_body(src_ref, dst_ref, vb, rsem, wsem):
    def _rd(q, step, p):
        blk = q * _CSTEPS + step
        return pltpu.make_async_copy(
            src_ref.at[pl.ds(blk * _CBLK, _CBLK)], vb.at[2 * q + p], rsem)

    def _wr(q, step, p):
        blk = q * _CSTEPS + step
        return pltpu.make_async_copy(
            vb.at[2 * q + p], dst_ref.at[pl.ds(blk * _CBLK, _CBLK)], wsem)

    for q in range(_NCHAIN):
        _rd(q, 0, 0).start()
    for step in range(_CSTEPS):
        p = step % 2
        for q in range(_NCHAIN):
            _rd(q, step, p).wait()
            if step >= 1:
                _wr(q, step - 1, 1 - p).wait()
            if step + 1 < _CSTEPS:
                _rd(q, step + 1, 1 - p).start()
            _wr(q, step, p).start()
    for q in range(_NCHAIN):
        _wr(q, _CSTEPS - 1, (_CSTEPS - 1) % 2).wait()


@jax.jit
def _tc_copy(buf):
    return pl.pallas_call(
        _copy_body,
        out_shape=jax.ShapeDtypeStruct((MEM, D), jnp.float32),
        in_specs=[pl.BlockSpec(memory_space=pl.ANY)],
        out_specs=pl.BlockSpec(memory_space=pl.ANY),
        scratch_shapes=[
            pltpu.VMEM((2 * _NCHAIN, _CBLK, D), jnp.float32),
            pltpu.SemaphoreType.DMA,
            pltpu.SemaphoreType.DMA,
        ],
    )(buf)


def _prep_body(img_in, x, y, idx, blab, ridx,
               w_out, nlab, rimg, rlab,
               idx_v, w_v, stage, ridx_v, lab_v, y_v, ridx_all, rlab_v,
               w_sh,
               sem, ssem):
    c = lax.axis_index("c")
    s = lax.axis_index("s")
    wid = s * NC + c
    lanes = lax.iota(jnp.int32, L)

    pltpu.sync_copy(idx, idx_v)

    @pl.when(s == 0)
    def _build_map():
        def mset(i, carry):
            w_v[pl.ds(i * L, L)] = jnp.full((L,), -1, jnp.int32)
            return carry

        lax.fori_loop(0, MEM // L, mset, 0)

        def setw(ci, carry):
            iv = idx_v[pl.ds(ci * L, L)]
            bids = ci * L + lanes
            for l in range(L):
                plsc.store_scatter(w_v, [iv], bids, mask=lanes == l)
            return carry

        lax.fori_loop(0, B // L, setw, 0)
        pltpu.sync_copy(w_v, w_sh)

        @pl.when(c == 0)
        def _publish():
            pltpu.sync_copy(w_v, w_out)

    plsc.subcore_barrier()
    pltpu.sync_copy(w_sh, w_v)

    rbase = wid * (R // NW)
    pltpu.sync_copy(ridx.at[pl.ds(rbase, R // NW)], ridx_v)
    rv = ridx_v[...]
    pltpu.async_copy(img_in.at[rv], stage.at[0], ssem).wait()
    pltpu.async_copy(stage.at[0], rimg.at[pl.ds(rbase, R // NW)], ssem).wait()
    wr = plsc.load_gather(w_v, [rv])
    updr = wr >= 0

    @pl.when(jnp.any(updr))
    def _ret_patch():
        f = plsc.all_reduce_ffs(updr)
        rv_f = plsc.load_gather(ridx_v, [f])
        wr_f = plsc.load_gather(w_v, [rv_f])
        srcs = jnp.where(updr, wr, wr_f)
        dst = jnp.where(updr, rbase + lanes, rbase + f)
        pltpu.async_copy(x.at[srcs], stage.at[1], ssem).wait()
        pltpu.async_copy(stage.at[1], rimg.at[dst], ssem).wait()

    @pl.when(jnp.logical_and(s == 0, c == 0))
    def _labels():
        pltpu.sync_copy(blab, lab_v)
        pltpu.sync_copy(y, y_v)

        def setl(ci, carry):
            iv = idx_v[pl.ds(ci * L, L)]
            yv = y_v[pl.ds(ci * L, L)]
            for l in range(L):
                plsc.store_scatter(lab_v, [iv], yv, mask=lanes == l)
            return carry

        lax.fori_loop(0, B // L, setl, 0)
        pltpu.sync_copy(lab_v, nlab)

        pltpu.sync_copy(ridx, ridx_all)

        def rl(ci, carry):
            rr = ridx_all[pl.ds(ci * L, L)]
            rlab_v[pl.ds(ci * L, L)] = plsc.load_gather(lab_v, [rr])
            return carry

        lax.fori_loop(0, R // L, rl, 0)
        pltpu.sync_copy(rlab_v, rlab)


_sc_prep = mpmd._mpmd_map(
    [(_MESH, _prep_body)],
    out_types=[
        jax.ShapeDtypeStruct((MEM,), jnp.int32),
        jax.ShapeDtypeStruct((MEM,), jnp.int32),
        jax.ShapeDtypeStruct((R, D), jnp.float32),
        jax.ShapeDtypeStruct((R,), jnp.int32),
    ],
    compiler_params=_SC_PARAMS,
    scratch_types=[
        pltpu.VMEM((B,), jnp.int32),
        pltpu.VMEM((MEM,), jnp.int32),
        pltpu.VMEM((2, L, D), jnp.float32),
        pltpu.VMEM((R // NW,), jnp.int32),
        pltpu.VMEM((MEM,), jnp.int32),
        pltpu.VMEM((B,), jnp.int32),
        pltpu.VMEM((R,), jnp.int32),
        pltpu.VMEM((R,), jnp.int32),
        pltpu.VMEM_SHARED((MEM,), jnp.int32),
        pltpu.SemaphoreType.DMA,
        pltpu.SemaphoreType.DMA,
    ],
)


def _scatter_body(img_in, x, idx, w_hbm,
                  img_out,
                  idx_v, w_v, stage, sem):
    c = lax.axis_index("c")
    s = lax.axis_index("s")
    wid = s * NC + c
    per = B // NW
    base = wid * per

    pltpu.sync_copy(w_hbm, w_v)
    pltpu.sync_copy(idx.at[pl.ds(base, per)], idx_v)
    for h in range(per // L):
        dv = idx_v[pl.ds(h * L, L)]
        srcs = plsc.load_gather(w_v, [dv])
        pltpu.async_copy(x.at[srcs], stage.at[h], sem).wait()
        pltpu.async_copy(stage.at[h], img_out.at[dv], sem).wait()


_sc_scatter = mpmd._mpmd_map(
    [(_MESH, _scatter_body)],
    out_types=[
        jax.ShapeDtypeStruct((MEM, D), jnp.float32),
    ],
    input_output_aliases={0: 0},
    compiler_params=_SC_PARAMS,
    scratch_types=[
        pltpu.VMEM((B // NW,), jnp.int32),
        pltpu.VMEM((MEM,), jnp.int32),
        pltpu.VMEM((2, L, D), jnp.float32),
        pltpu.SemaphoreType.DMA,
    ],
)


def kernel(buffer_img, buffer_label, x, y, idx, retrieve_idx):
    img2 = buffer_img.reshape(MEM, D)
    x2 = x.reshape(B, D)
    y32 = y.astype(jnp.int32)
    idx32 = idx.astype(jnp.int32)
    ridx32 = retrieve_idx.astype(jnp.int32)
    blab32 = buffer_label.astype(jnp.int32)

    w, new_label, ret_img2, ret_label = _sc_prep(
        img2, x2, y32, idx32, blab32, ridx32)
    img0 = _tc_copy(img2)
    (new_img2,) = _sc_scatter(img0, x2, idx32, w)

    new_img = new_img2.reshape(MEM, 3, 32, 32)
    ret_img = ret_img2.reshape(R, 3, 32, 32)
    return (new_img,
            new_label.astype(buffer_label.dtype),
            ret_img,
            ret_label.astype(buffer_label.dtype))


# speedup vs baseline: 1.0263x; 1.0028x over previous
import jax
import jax.numpy as jnp
from jax import lax
from jax.experimental import pallas as pl
from jax.experimental.pallas import tpu as pltpu
from jax.experimental.pallas import tpu_sc as plsc
from jax._src.pallas import mpmd

MEM = 10000
D = 3072
B = 1024
R = 512
NC = 2
NS = 16
NW = NC * NS
L = 16

_MESH = plsc.VectorSubcoreMesh(core_axis_name="c", subcore_axis_name="s")
_SC_PARAMS = pltpu.CompilerParams(needs_layout_passes=False)


_NCHAIN = 5
_CSTEPS = 10
_CBLK = MEM // (_NCHAIN * _CSTEPS)


def _copy_body(src_ref, dst_ref, vb, rsem, wsem):
    def _rd(q, step, p):
        blk = q * _CSTEPS + step
        return pltpu.make_async_copy(
            src_ref.at[pl.ds(blk * _CBLK, _CBLK)], vb.at[2 * q + p], rsem)

    def _wr(q, step, p):
        blk = q * _CSTEPS + step
        return pltpu.make_async_copy(
            vb.at[2 * q + p], dst_ref.at[pl.ds(blk * _CBLK, _CBLK)], wsem)

    for q in range(_NCHAIN):
        _rd(q, 0, 0).start()
    for step in range(_CSTEPS):
        p = step % 2
        for q in range(_NCHAIN):
            _rd(q, step, p).wait()
            if step >= 1:
                _wr(q, step - 1, 1 - p).wait()
            if step + 1 < _CSTEPS:
                _rd(q, step + 1, 1 - p).start()
            _wr(q, step, p).start()
    for q in range(_NCHAIN):
        _wr(q, _CSTEPS - 1, (_CSTEPS - 1) % 2).wait()


@jax.jit
def _tc_copy(buf):
    return pl.pallas_call(
        _copy_body,
        out_shape=jax.ShapeDtypeStruct((MEM, D), jnp.float32),
        in_specs=[pl.BlockSpec(memory_space=pl.ANY)],
        out_specs=pl.BlockSpec(memory_space=pl.ANY),
        scratch_shapes=[
            pltpu.VMEM((2 * _NCHAIN, _CBLK, D), jnp.float32),
            pltpu.SemaphoreType.DMA,
            pltpu.SemaphoreType.DMA,
        ],
    )(buf)


def _prep_body(img_in, x, y, idx, blab, ridx,
               w_out, nlab, rimg, rlab,
               idx_v, w_v, stage, ridx_v, lab_v, y_v, ridx_all, rlab_v,
               w_sh,
               sem, ssem):
    c = lax.axis_index("c")
    s = lax.axis_index("s")
    wid = s * NC + c
    lanes = lax.iota(jnp.int32, L)

    pltpu.sync_copy(idx, idx_v)

    @pl.when(s == 0)
    def _build_map():
        def mset(i, carry):
            w_v[pl.ds(i * L, L)] = jnp.full((L,), -1, jnp.int32)
            return carry

        lax.fori_loop(0, MEM // L, mset, 0)

        def setw(ci, carry):
            iv = idx_v[pl.ds(ci * L, L)]
            bids = ci * L + lanes
            for l in range(L):
                plsc.store_scatter(w_v, [iv], bids, mask=lanes == l)
            return carry

        lax.fori_loop(0, B // L, setw, 0)
        pltpu.sync_copy(w_v, w_sh)

        @pl.when(c == 0)
        def _publish():
            pltpu.sync_copy(w_v, w_out)

    plsc.subcore_barrier()
    pltpu.sync_copy(w_sh, w_v)

    rbase = wid * (R // NW)
    pltpu.sync_copy(ridx.at[pl.ds(rbase, R // NW)], ridx_v)
    rv = ridx_v[...]
    pltpu.async_copy(img_in.at[rv], stage.at[0], ssem).wait()
    pltpu.async_copy(stage.at[0], rimg.at[pl.ds(rbase, R // NW)], ssem).wait()
    wr = plsc.load_gather(w_v, [rv])
    updr = wr >= 0

    @pl.when(jnp.any(updr))
    def _ret_patch():
        f = plsc.all_reduce_ffs(updr)
        rv_f = plsc.load_gather(ridx_v, [f])
        wr_f = plsc.load_gather(w_v, [rv_f])
        srcs = jnp.where(updr, wr, wr_f)
        dst = jnp.where(updr, rbase + lanes, rbase + f)
        pltpu.async_copy(x.at[srcs], stage.at[1], ssem).wait()
        pltpu.async_copy(stage.at[1], rimg.at[dst], ssem).wait()

    @pl.when(jnp.logical_and(s == 0, c == 0))
    def _labels():
        pltpu.sync_copy(blab, lab_v)
        pltpu.sync_copy(y, y_v)

        def setl(ci, carry):
            iv = idx_v[pl.ds(ci * L, L)]
            yv = y_v[pl.ds(ci * L, L)]
            for l in range(L):
                plsc.store_scatter(lab_v, [iv], yv, mask=lanes == l)
            return carry

        lax.fori_loop(0, B // L, setl, 0)
        pltpu.sync_copy(lab_v, nlab)

        pltpu.sync_copy(ridx, ridx_all)

        def rl(ci, carry):
            rr = ridx_all[pl.ds(ci * L, L)]
            rlab_v[pl.ds(ci * L, L)] = plsc.load_gather(lab_v, [rr])
            return carry

        lax.fori_loop(0, R // L, rl, 0)
        pltpu.sync_copy(rlab_v, rlab)


_sc_prep = mpmd._mpmd_map(
    [(_MESH, _prep_body)],
    out_types=[
        jax.ShapeDtypeStruct((MEM,), jnp.int32),
        jax.ShapeDtypeStruct((MEM,), jnp.int32),
        jax.ShapeDtypeStruct((R, D), jnp.float32),
        jax.ShapeDtypeStruct((R,), jnp.int32),
    ],
    compiler_params=_SC_PARAMS,
    scratch_types=[
        pltpu.VMEM((B,), jnp.int32),
        pltpu.VMEM((MEM,), jnp.int32),
        pltpu.VMEM((2, L, D), jnp.float32),
        pltpu.VMEM((R // NW,), jnp.int32),
        pltpu.VMEM((MEM,), jnp.int32),
        pltpu.VMEM((B,), jnp.int32),
        pltpu.VMEM((R,), jnp.int32),
        pltpu.VMEM((R,), jnp.int32),
        pltpu.VMEM_SHARED((MEM,), jnp.int32),
        pltpu.SemaphoreType.DMA,
        pltpu.SemaphoreType.DMA,
    ],
)


def _scatter_body(img_in, x, idx, w_hbm,
                  img_out,
                  idx_v, w_v, stage, sem, wsem):
    c = lax.axis_index("c")
    s = lax.axis_index("s")
    wid = s * NC + c
    per = B // NW
    base = wid * per

    pltpu.sync_copy(w_hbm, w_v)
    pltpu.sync_copy(idx.at[pl.ds(base, per)], idx_v)
    dvs, gathers = [], []
    for h in range(per // L):
        dv = idx_v[pl.ds(h * L, L)]
        srcs = plsc.load_gather(w_v, [dv])
        dvs.append(dv)
        gathers.append(pltpu.async_copy(x.at[srcs], stage.at[h], sem))
    scatters = []
    for h in range(per // L):
        gathers[h].wait()
        scatters.append(
            pltpu.async_copy(stage.at[h], img_out.at[dvs[h]], wsem))
    for sc in scatters:
        sc.wait()


_sc_scatter = mpmd._mpmd_map(
    [(_MESH, _scatter_body)],
    out_types=[
        jax.ShapeDtypeStruct((MEM, D), jnp.float32),
    ],
    input_output_aliases={0: 0},
    compiler_params=_SC_PARAMS,
    scratch_types=[
        pltpu.VMEM((B // NW,), jnp.int32),
        pltpu.VMEM((MEM,), jnp.int32),
        pltpu.VMEM((2, L, D), jnp.float32),
        pltpu.SemaphoreType.DMA,
        pltpu.SemaphoreType.DMA,
    ],
)


def kernel(buffer_img, buffer_label, x, y, idx, retrieve_idx):
    img2 = buffer_img.reshape(MEM, D)
    x2 = x.reshape(B, D)
    y32 = y.astype(jnp.int32)
    idx32 = idx.astype(jnp.int32)
    ridx32 = retrieve_idx.astype(jnp.int32)
    blab32 = buffer_label.astype(jnp.int32)

    w, new_label, ret_img2, ret_label = _sc_prep(
        img2, x2, y32, idx32, blab32, ridx32)
    img0 = _tc_copy(img2)
    (new_img2,) = _sc_scatter(img0, x2, idx32, w)

    new_img = new_img2.reshape(MEM, 3, 32, 32)
    ret_img = ret_img2.reshape(R, 3, 32, 32)
    return (new_img,
            new_label.astype(buffer_label.dtype),
            ret_img,
            ret_label.astype(buffer_label.dtype))
